# baseline (speedup 1.0000x reference)
import jax
import jax.numpy as jnp
from jax import lax
from jax.experimental import pallas as pl
from jax.experimental.pallas import tpu as pltpu
from jax.experimental.pallas import tpu_sc as plsc

L = 16
DIM = 64
BATCH = 16384
NUM_CORES = 2
NUM_SUBCORES = 16
NW = NUM_CORES * NUM_SUBCORES
B_PER_W = BATCH // NW
NROWS = 1000000
BKT_W = 128
NBKT_TAB = 7840
NBKT = 2 * NBKT_TAB
R_PER_W = NBKT // NW
TASK_CAP = 1536
CHUNK = 2048
BANK = 64
RING = 2 * BANK


def _splat(x):
    return jnp.full((L,), x, jnp.int32)


def _scalar(v):
    return jnp.squeeze(lax.slice(v, (0,), (1,)))


def _gather_body(uidx_hbm, iidx_hbm, utab_hbm, itab_hbm, rowstage_hbm,
                 idx_v, tval_v, tbkt_v, teid_v, sval_v, seid_v,
                 hist_v, bstart_v, cursor_v, blist_v,
                 blk0_v, blk1_v, blk2_v, blk3_v, blk4_v, blk5_v, blk6_v,
                 blk7_v, blk8_v, blk9_v, blk10_v, blk11_v, rowbuf_v,
                 sem_b0, sem_b1, sem_b2, sem_b3, sem_b4,
                 sem_b5, sem_b6, sem_b7, sem_b8, sem_b9, sem_b10, sem_b11,
                 sem_ra, sem_rb):
    wid = lax.axis_index("s") * NUM_CORES + lax.axis_index("c")
    lo = wid * R_PER_W
    lane = lax.iota(jnp.int32, L)
    lo_v = _splat(lo)
    hi_v = _splat(lo + R_PER_W)
    ones = _splat(1)

    def scan_one(idx_hbm, eid_base, off0):
        def chunk_body(c, off_vec):
            pltpu.sync_copy(idx_hbm.at[pl.ds(c * CHUNK, CHUNK)], idx_v)

            def vec_body(j, off_vec):
                vals = idx_v[pl.ds(j * L, L)]
                g = eid_base[1] + lax.shift_right_logical(vals, 7)
                m = jnp.logical_and(g >= lo_v, g < hi_v)
                m32 = m.astype(jnp.int32)
                pos = off_vec + plsc.cumsum(m32) - m32
                eid = _splat(eid_base[0] + c * CHUNK + j * L) + lane
                plsc.store_scatter(tval_v, [pos], vals, mask=m)
                plsc.store_scatter(tbkt_v, [pos], g - lo_v, mask=m)
                plsc.store_scatter(teid_v, [pos], eid, mask=m)
                return off_vec + plsc.all_reduce_population_count(m)

            return lax.fori_loop(0, CHUNK // L, vec_body, off_vec)

        return lax.fori_loop(0, BATCH // CHUNK, chunk_body, off0)

    off = scan_one(uidx_hbm, (0, _splat(0)), _splat(0))
    off = scan_one(iidx_hbm, (BATCH, _splat(NBKT_TAB)), off)

    zero = jnp.zeros((L,), jnp.int32)
    for k in range(512 // L):
        hist_v[pl.ds(k * L, L)] = zero

    n_task = _scalar(off)
    n_chunks = (n_task + L - 1) // L

    def hist_body(k, _):
        tm = (_splat(k * L) + lane) < off
        bl = tbkt_v[pl.ds(k * L, L)]
        plsc.addupdate_scatter(hist_v, [bl], ones, mask=tm)
        return 0

    lax.fori_loop(0, n_chunks, hist_body, 0)

    def prefix_body(k, carry):
        carry_sum, nb_vec = carry
        c = hist_v[pl.ds(k * L, L)]
        s = plsc.cumsum(c)
        excl = s - c + carry_sum
        bstart_v[pl.ds(k * L, L)] = excl
        cursor_v[pl.ds(k * L, L)] = excl
        total = jnp.take_along_axis(s, _splat(L - 1), axis=0)
        nz = c > 0
        nz32 = nz.astype(jnp.int32)
        posb = nb_vec + plsc.cumsum(nz32) - nz32
        plsc.store_scatter(blist_v, [posb], _splat(k * L) + lane, mask=nz)
        return (carry_sum + total, nb_vec + plsc.all_reduce_population_count(nz))

    carry_sum, nb_vec = lax.fori_loop(
        0, 512 // L, prefix_body, (_splat(0), _splat(0)))
    nb = _scalar(nb_vec)

    def scatter_chunk(k, _):
        gidx = _splat(k * L) + lane
        tm = gidx < off
        tm32 = tm.astype(jnp.int32)
        bl = tbkt_v[pl.ds(k * L, L)]
        tv = tval_v[pl.ds(k * L, L)]
        te = teid_v[pl.ds(k * L, L)]
        base = plsc.load_gather(cursor_v, [bl], mask=tm)
        rank = jnp.zeros((L,), jnp.int32)
        eqt = jnp.zeros((L,), jnp.int32)
        for kk in range(L):
            pk = jnp.take_along_axis(bl, _splat(kk), axis=0)
            vk = jnp.take_along_axis(tm32, _splat(kk), axis=0)
            eqk = (pk == bl).astype(jnp.int32) * vk
            rank = rank + jnp.where(lane > kk, eqk, 0)
            eqt = eqt + eqk
        p = base + rank
        plsc.store_scatter(sval_v, [p], tv, mask=tm)
        plsc.store_scatter(seid_v, [p], te, mask=tm)
        lastm = jnp.logical_and(tm, rank == (eqt - 1))
        plsc.store_scatter(cursor_v, [bl], base + eqt, mask=lastm)
        return 0

    lax.fori_loop(0, n_chunks, scatter_chunk, 0)

    blks = (blk0_v, blk1_v, blk2_v, blk3_v, blk4_v, blk5_v, blk6_v, blk7_v,
            blk8_v, blk9_v, blk10_v, blk11_v)
    sems = (sem_b0, sem_b1, sem_b2, sem_b3, sem_b4, sem_b5, sem_b6, sem_b7,
            sem_b8, sem_b9, sem_b10, sem_b11)
    BLK_N = len(blks)

    def _rd(ref, idx):
        return ref[pl.ds(idx, L)][0]

    def _drain(sem, cnt):
        def body(_, __):
            pltpu.make_async_copy(
                rowstage_hbm.at[pl.ds(0, DIM)],
                rowbuf_v.at[pl.ds(0, DIM)], sem).wait()
            return 0

        lax.fori_loop(0, cnt, body, 0)

    def issue_blk(k, blkref, sem):
        bl = _rd(blist_v, k)
        g = lo + bl
        is_item = (g >= NBKT_TAB).astype(jnp.int32)
        col0 = (g - is_item * NBKT_TAB) * BKT_W
        col0 = pl.multiple_of(col0, BKT_W)

        @pl.when(is_item == 0)
        def _():
            pltpu.async_copy(utab_hbm.at[:, pl.ds(col0, BKT_W)], blkref, sem)

        @pl.when(is_item == 1)
        def _():
            pltpu.async_copy(itab_hbm.at[:, pl.ds(col0, BKT_W)], blkref, sem)

    for j in range(BLK_N - 1):
        @pl.when(j < nb)
        def _(j=j):
            issue_blk(j, blks[j], sems[j])

    def process_blk(k, s):
        pre = (s + BLK_N - 1) % BLK_N

        @pl.when(k + BLK_N - 1 < nb)
        def _():
            issue_blk(k + BLK_N - 1, blks[pre], sems[pre])

        pltpu.make_async_copy(
            utab_hbm.at[:, pl.ds(0, BKT_W)], blks[s], sems[s]).wait()

        bl = _rd(blist_v, k)
        g = lo + bl
        is_item = (g >= NBKT_TAB).astype(jnp.int32)
        col0 = (g - is_item * NBKT_TAB) * BKT_W
        start = _rd(bstart_v, bl)
        end = _rd(cursor_v, bl)

        def task_body(t, _):
            slot = lax.rem(t, RING)
            @pl.when(jnp.logical_and(slot == 0, t >= RING))
            def _():
                _drain(sem_ra, BANK)

            @pl.when(jnp.logical_and(slot == BANK, t >= RING))
            def _():
                _drain(sem_rb, BANK)

            val = _rd(sval_v, t)
            eid = _rd(seid_v, t)
            l_v = _splat(val - col0)
            r = pl.multiple_of(slot * DIM, DIM)
            for q in range(DIM // L):
                gq = plsc.load_gather(blks[s], [_splat(q * L) + lane, l_v])
                rowbuf_v[pl.ds(r + q * L, L)] = gq

            dst = pl.multiple_of(eid * DIM, DIM)

            @pl.when(slot < BANK)
            def _():
                pltpu.async_copy(rowbuf_v.at[pl.ds(r, DIM)],
                                 rowstage_hbm.at[pl.ds(dst, DIM)], sem_ra)

            @pl.when(slot >= BANK)
            def _():
                pltpu.async_copy(rowbuf_v.at[pl.ds(r, DIM)],
                                 rowstage_hbm.at[pl.ds(dst, DIM)], sem_rb)
            return 0

        lax.fori_loop(start, end, task_body, 0)

    def ring_body(m, _):
        for s in range(BLK_N):
            k = BLK_N * m + s

            @pl.when(k < nb)
            def _(k=k, s=s):
                process_blk(k, s)
        return 0

    lax.fori_loop(0, (nb + BLK_N - 1) // BLK_N, ring_body, 0)

    n = n_task
    cycles = n // RING
    rem = lax.rem(n, RING)
    issued_a = cycles * BANK + jnp.minimum(rem, BANK)
    issued_b = cycles * BANK + jnp.maximum(rem - BANK, 0)
    drained_a = BANK * jnp.maximum((n - 1) // RING, 0)
    drained_b = BANK * jnp.maximum((n - 1 - BANK) // RING, 0)
    _drain(sem_ra, issued_a - drained_a)
    _drain(sem_rb, issued_b - drained_b)


def _dot_body(rowstage_hbm, out_hbm, uslab_v, islab_v, out_v):
    wid = lax.axis_index("s") * NUM_CORES + lax.axis_index("c")
    base = wid * B_PER_W
    pltpu.sync_copy(rowstage_hbm.at[pl.ds(base * DIM, B_PER_W * DIM)], uslab_v)
    pltpu.sync_copy(
        rowstage_hbm.at[pl.ds((BATCH + base) * DIM, B_PER_W * DIM)], islab_v)

    lane = lax.iota(jnp.int32, L)
    perms = [lane ^ (L >> (k + 1)) for k in range(4)]

    def group_body(g, _):
        acc = jnp.zeros((L,), jnp.float32)
        for r in range(L):
            row = g * L + r
            s = None
            for c in range(DIM // L):
                up = uslab_v[pl.ds(row * DIM + c * L, L)]
                ip = islab_v[pl.ds(row * DIM + c * L, L)]
                p = up * ip
                s = p if s is None else s + p
            for p_idx in perms:
                s = s + jnp.take_along_axis(s, p_idx, axis=0)
            acc = jnp.where(lane == r, s, acc)
        rating = 1.0 / (1.0 + jnp.exp(-acc))
        out_v[pl.ds(g * L, L)] = rating
        return 0

    lax.fori_loop(0, B_PER_W // L, group_body, 0)
    pltpu.sync_copy(out_v, out_hbm.at[pl.ds(base, B_PER_W)])


@jax.jit
def _wmf(user_indices, item_indices, user_table, item_table):
    mesh = plsc.VectorSubcoreMesh(core_axis_name="c", subcore_axis_name="s")
    rowstage = pl.kernel(
        _gather_body,
        out_type=jax.ShapeDtypeStruct((2 * BATCH * DIM,), jnp.float32),
        mesh=mesh,
        compiler_params=pltpu.CompilerParams(needs_layout_passes=False),
        scratch_types=[
            pltpu.VMEM((CHUNK,), jnp.int32),
            pltpu.VMEM((TASK_CAP,), jnp.int32),
            pltpu.VMEM((TASK_CAP,), jnp.int32),
            pltpu.VMEM((TASK_CAP,), jnp.int32),
            pltpu.VMEM((TASK_CAP + L,), jnp.int32),
            pltpu.VMEM((TASK_CAP + L,), jnp.int32),
            pltpu.VMEM((512,), jnp.int32),
            pltpu.VMEM((512 + L,), jnp.int32),
            pltpu.VMEM((512 + L,), jnp.int32),
            pltpu.VMEM((512 + L,), jnp.int32),
            pltpu.VMEM((DIM, BKT_W), jnp.float32),
            pltpu.VMEM((DIM, BKT_W), jnp.float32),
            pltpu.VMEM((DIM, BKT_W), jnp.float32),
            pltpu.VMEM((DIM, BKT_W), jnp.float32),
            pltpu.VMEM((DIM, BKT_W), jnp.float32),
            pltpu.VMEM((DIM, BKT_W), jnp.float32),
            pltpu.VMEM((DIM, BKT_W), jnp.float32),
            pltpu.VMEM((DIM, BKT_W), jnp.float32),
            pltpu.VMEM((DIM, BKT_W), jnp.float32),
            pltpu.VMEM((DIM, BKT_W), jnp.float32),
            pltpu.VMEM((DIM, BKT_W), jnp.float32),
            pltpu.VMEM((DIM, BKT_W), jnp.float32),
            pltpu.VMEM((RING * DIM,), jnp.float32),
            pltpu.SemaphoreType.DMA,
            pltpu.SemaphoreType.DMA,
            pltpu.SemaphoreType.DMA,
            pltpu.SemaphoreType.DMA,
            pltpu.SemaphoreType.DMA,
            pltpu.SemaphoreType.DMA,
            pltpu.SemaphoreType.DMA,
            pltpu.SemaphoreType.DMA,
            pltpu.SemaphoreType.DMA,
            pltpu.SemaphoreType.DMA,
            pltpu.SemaphoreType.DMA,
            pltpu.SemaphoreType.DMA,
            pltpu.SemaphoreType.DMA,
            pltpu.SemaphoreType.DMA,
        ],
    )(user_indices, item_indices, user_table.T, item_table.T)

    return pl.kernel(
        _dot_body,
        out_type=jax.ShapeDtypeStruct((BATCH,), jnp.float32),
        mesh=mesh,
        scratch_types=[
            pltpu.VMEM((B_PER_W * DIM,), jnp.float32),
            pltpu.VMEM((B_PER_W * DIM,), jnp.float32),
            pltpu.VMEM((B_PER_W,), jnp.float32),
        ],
    )(rowstage)


def kernel(user_indices, item_indices, user_table, item_table):
    return _wmf(user_indices.astype(jnp.int32), item_indices.astype(jnp.int32),
                user_table, item_table)


# speedup vs baseline: 1.0884x; 1.0884x over previous
import jax
import jax.numpy as jnp
from jax import lax
from jax.experimental import pallas as pl
from jax.experimental.pallas import tpu as pltpu
from jax.experimental.pallas import tpu_sc as plsc

L = 16
DIM = 64
BATCH = 16384
NUM_CORES = 2
NUM_SUBCORES = 16
NW = NUM_CORES * NUM_SUBCORES
B_PER_W = BATCH // NW
NROWS = 1000000
BKT_W = 128
NBKT_TAB = 7840
NBKT = 2 * NBKT_TAB
R_PER_W = NBKT // NW
TASK_CAP = 1536
CHUNK = 2048
BANK = 64
RING = 2 * BANK


def _splat(x):
    return jnp.full((L,), x, jnp.int32)


def _scalar(v):
    return jnp.squeeze(lax.slice(v, (0,), (1,)))


def _gather_body(uidx_hbm, iidx_hbm, utab_hbm, itab_hbm, rowstage_hbm,
                 idx_v, idx2_v, tval_v, tbkt_v, teid_v, sval_v, seid_v,
                 hist_v, bstart_v, cursor_v, blist_v,
                 blk0_v, blk1_v, blk2_v, blk3_v, blk4_v, blk5_v, blk6_v,
                 blk7_v, rowbuf_v, sem_b0, sem_b1, sem_b2, sem_b3, sem_b4,
                 sem_b5, sem_b6, sem_b7, sem_ra, sem_rb, sem_i0, sem_i1):
    wid = lax.axis_index("s") * NUM_CORES + lax.axis_index("c")
    lo = wid * R_PER_W
    lane = lax.iota(jnp.int32, L)
    lo_v = _splat(lo)
    hi_v = _splat(lo + R_PER_W)
    ones = _splat(1)

    items = [(uidx_hbm, c, 0, 0) for c in range(BATCH // CHUNK)]
    items += [(iidx_hbm, c, BATCH, NBKT_TAB) for c in range(BATCH // CHUNK)]
    ibufs = (idx_v, idx2_v)
    isems = (sem_i0, sem_i1)
    pltpu.async_copy(items[0][0].at[pl.ds(0, CHUNK)], ibufs[0], isems[0])
    off = _splat(0)
    for n, (idx_hbm, c, eb, bb) in enumerate(items):
        s = n % 2
        if n + 1 < len(items):
            h2, c2 = items[n + 1][0], items[n + 1][1]
            pltpu.async_copy(h2.at[pl.ds(c2 * CHUNK, CHUNK)],
                             ibufs[1 - s], isems[1 - s])
        pltpu.make_async_copy(
            idx_hbm.at[pl.ds(0, CHUNK)], ibufs[s], isems[s]).wait()
        bb_v = _splat(bb)

        def vec_body(j, off_vec, s=s, c=c, eb=eb, bb_v=bb_v):
            vals = ibufs[s][pl.ds(j * L, L)]
            g = bb_v + lax.shift_right_logical(vals, 7)
            m = jnp.logical_and(g >= lo_v, g < hi_v)
            m32 = m.astype(jnp.int32)
            pos = off_vec + plsc.cumsum(m32) - m32
            eid = _splat(eb + c * CHUNK + j * L) + lane
            plsc.store_scatter(tval_v, [pos], vals, mask=m)
            plsc.store_scatter(tbkt_v, [pos], g - lo_v, mask=m)
            plsc.store_scatter(teid_v, [pos], eid, mask=m)
            return off_vec + plsc.all_reduce_population_count(m)

        off = lax.fori_loop(0, CHUNK // L, vec_body, off)

    zero = jnp.zeros((L,), jnp.int32)
    for k in range(512 // L):
        hist_v[pl.ds(k * L, L)] = zero

    n_task = _scalar(off)
    n_chunks = (n_task + L - 1) // L

    def hist_body(k, _):
        tm = (_splat(k * L) + lane) < off
        bl = tbkt_v[pl.ds(k * L, L)]
        plsc.addupdate_scatter(hist_v, [bl], ones, mask=tm)
        return 0

    lax.fori_loop(0, n_chunks, hist_body, 0)

    def prefix_body(k, carry):
        carry_sum, nb_vec = carry
        c = hist_v[pl.ds(k * L, L)]
        s = plsc.cumsum(c)
        excl = s - c + carry_sum
        bstart_v[pl.ds(k * L, L)] = excl
        cursor_v[pl.ds(k * L, L)] = excl
        total = jnp.take_along_axis(s, _splat(L - 1), axis=0)
        nz = c > 0
        nz32 = nz.astype(jnp.int32)
        posb = nb_vec + plsc.cumsum(nz32) - nz32
        plsc.store_scatter(blist_v, [posb], _splat(k * L) + lane, mask=nz)
        return (carry_sum + total, nb_vec + plsc.all_reduce_population_count(nz))

    carry_sum, nb_vec = lax.fori_loop(
        0, 512 // L, prefix_body, (_splat(0), _splat(0)))
    nb = _scalar(nb_vec)

    def scatter_chunk(k, _):
        gidx = _splat(k * L) + lane
        tm = gidx < off
        tm32 = tm.astype(jnp.int32)
        bl = tbkt_v[pl.ds(k * L, L)]
        tv = tval_v[pl.ds(k * L, L)]
        te = teid_v[pl.ds(k * L, L)]
        base = plsc.load_gather(cursor_v, [bl], mask=tm)
        rank = jnp.zeros((L,), jnp.int32)
        eqt = jnp.zeros((L,), jnp.int32)
        for kk in range(L):
            pk = jnp.take_along_axis(bl, _splat(kk), axis=0)
            vk = jnp.take_along_axis(tm32, _splat(kk), axis=0)
            eqk = (pk == bl).astype(jnp.int32) * vk
            rank = rank + jnp.where(lane > kk, eqk, 0)
            eqt = eqt + eqk
        p = base + rank
        plsc.store_scatter(sval_v, [p], tv, mask=tm)
        plsc.store_scatter(seid_v, [p], te, mask=tm)
        lastm = jnp.logical_and(tm, rank == (eqt - 1))
        plsc.store_scatter(cursor_v, [bl], base + eqt, mask=lastm)
        return 0

    lax.fori_loop(0, n_chunks, scatter_chunk, 0)

    blks = (blk0_v, blk1_v, blk2_v, blk3_v, blk4_v, blk5_v, blk6_v, blk7_v)
    sems = (sem_b0, sem_b1, sem_b2, sem_b3, sem_b4, sem_b5, sem_b6, sem_b7)
    BLK_N = len(blks)

    def _rd(ref, idx):
        return ref[pl.ds(idx, L)][0]

    def _drain(sem, cnt):
        def body(_, __):
            pltpu.make_async_copy(
                rowstage_hbm.at[pl.ds(0, DIM)],
                rowbuf_v.at[pl.ds(0, DIM)], sem).wait()
            return 0

        lax.fori_loop(0, cnt, body, 0)

    def issue_blk(k, blkref, sem):
        bl = _rd(blist_v, k)
        g = lo + bl
        is_item = (g >= NBKT_TAB).astype(jnp.int32)
        col0 = (g - is_item * NBKT_TAB) * BKT_W
        col0 = pl.multiple_of(col0, BKT_W)

        @pl.when(is_item == 0)
        def _():
            pltpu.async_copy(utab_hbm.at[:, pl.ds(col0, BKT_W)], blkref, sem)

        @pl.when(is_item == 1)
        def _():
            pltpu.async_copy(itab_hbm.at[:, pl.ds(col0, BKT_W)], blkref, sem)

    for j in range(BLK_N - 1):
        @pl.when(j < nb)
        def _(j=j):
            issue_blk(j, blks[j], sems[j])

    def process_blk(k, s):
        pre = (s + BLK_N - 1) % BLK_N

        @pl.when(k + BLK_N - 1 < nb)
        def _():
            issue_blk(k + BLK_N - 1, blks[pre], sems[pre])

        pltpu.make_async_copy(
            utab_hbm.at[:, pl.ds(0, BKT_W)], blks[s], sems[s]).wait()

        bl = _rd(blist_v, k)
        g = lo + bl
        is_item = (g >= NBKT_TAB).astype(jnp.int32)
        col0 = (g - is_item * NBKT_TAB) * BKT_W
        start = _rd(bstart_v, bl)
        end = _rd(cursor_v, bl)

        def task_body(t, _):
            slot = lax.rem(t, RING)
            @pl.when(jnp.logical_and(slot == 0, t >= RING))
            def _():
                _drain(sem_ra, BANK)

            @pl.when(jnp.logical_and(slot == BANK, t >= RING))
            def _():
                _drain(sem_rb, BANK)

            val = _rd(sval_v, t)
            eid = _rd(seid_v, t)
            l_v = _splat(val - col0)
            r = pl.multiple_of(slot * DIM, DIM)
            for q in range(DIM // L):
                gq = plsc.load_gather(blks[s], [_splat(q * L) + lane, l_v])
                rowbuf_v[pl.ds(r + q * L, L)] = gq

            dst = pl.multiple_of(eid * DIM, DIM)

            @pl.when(slot < BANK)
            def _():
                pltpu.async_copy(rowbuf_v.at[pl.ds(r, DIM)],
                                 rowstage_hbm.at[pl.ds(dst, DIM)], sem_ra)

            @pl.when(slot >= BANK)
            def _():
                pltpu.async_copy(rowbuf_v.at[pl.ds(r, DIM)],
                                 rowstage_hbm.at[pl.ds(dst, DIM)], sem_rb)
            return 0

        lax.fori_loop(start, end, task_body, 0)

    def ring_body(m, _):
        for s in range(BLK_N):
            k = BLK_N * m + s

            @pl.when(k < nb)
            def _(k=k, s=s):
                process_blk(k, s)
        return 0

    lax.fori_loop(0, (nb + BLK_N - 1) // BLK_N, ring_body, 0)

    n = n_task
    cycles = n // RING
    rem = lax.rem(n, RING)
    issued_a = cycles * BANK + jnp.minimum(rem, BANK)
    issued_b = cycles * BANK + jnp.maximum(rem - BANK, 0)
    drained_a = BANK * jnp.maximum((n - 1) // RING, 0)
    drained_b = BANK * jnp.maximum((n - 1 - BANK) // RING, 0)
    _drain(sem_ra, issued_a - drained_a)
    _drain(sem_rb, issued_b - drained_b)


def _dot_body(rowstage_hbm, out_hbm, uslab_v, islab_v, out_v):
    wid = lax.axis_index("s") * NUM_CORES + lax.axis_index("c")
    base = wid * B_PER_W
    pltpu.sync_copy(rowstage_hbm.at[pl.ds(base * DIM, B_PER_W * DIM)], uslab_v)
    pltpu.sync_copy(
        rowstage_hbm.at[pl.ds((BATCH + base) * DIM, B_PER_W * DIM)], islab_v)

    lane = lax.iota(jnp.int32, L)
    perms = [lane ^ (L >> (k + 1)) for k in range(4)]

    def group_body(g, _):
        acc = jnp.zeros((L,), jnp.float32)
        for r in range(L):
            row = g * L + r
            s = None
            for c in range(DIM // L):
                up = uslab_v[pl.ds(row * DIM + c * L, L)]
                ip = islab_v[pl.ds(row * DIM + c * L, L)]
                p = up * ip
                s = p if s is None else s + p
            for p_idx in perms:
                s = s + jnp.take_along_axis(s, p_idx, axis=0)
            acc = jnp.where(lane == r, s, acc)
        rating = 1.0 / (1.0 + jnp.exp(-acc))
        out_v[pl.ds(g * L, L)] = rating
        return 0

    lax.fori_loop(0, B_PER_W // L, group_body, 0)
    pltpu.sync_copy(out_v, out_hbm.at[pl.ds(base, B_PER_W)])


@jax.jit
def _wmf(user_indices, item_indices, user_table, item_table):
    mesh = plsc.VectorSubcoreMesh(core_axis_name="c", subcore_axis_name="s")
    rowstage = pl.kernel(
        _gather_body,
        out_type=jax.ShapeDtypeStruct((2 * BATCH * DIM,), jnp.float32),
        mesh=mesh,
        compiler_params=pltpu.CompilerParams(needs_layout_passes=False),
        scratch_types=[
            pltpu.VMEM((CHUNK,), jnp.int32),
            pltpu.VMEM((CHUNK,), jnp.int32),
            pltpu.VMEM((TASK_CAP,), jnp.int32),
            pltpu.VMEM((TASK_CAP,), jnp.int32),
            pltpu.VMEM((TASK_CAP,), jnp.int32),
            pltpu.VMEM((TASK_CAP + L,), jnp.int32),
            pltpu.VMEM((TASK_CAP + L,), jnp.int32),
            pltpu.VMEM((512,), jnp.int32),
            pltpu.VMEM((512 + L,), jnp.int32),
            pltpu.VMEM((512 + L,), jnp.int32),
            pltpu.VMEM((512 + L,), jnp.int32),
            pltpu.VMEM((DIM, BKT_W), jnp.float32),
            pltpu.VMEM((DIM, BKT_W), jnp.float32),
            pltpu.VMEM((DIM, BKT_W), jnp.float32),
            pltpu.VMEM((DIM, BKT_W), jnp.float32),
            pltpu.VMEM((DIM, BKT_W), jnp.float32),
            pltpu.VMEM((DIM, BKT_W), jnp.float32),
            pltpu.VMEM((DIM, BKT_W), jnp.float32),
            pltpu.VMEM((DIM, BKT_W), jnp.float32),
            pltpu.VMEM((RING * DIM,), jnp.float32),
            pltpu.SemaphoreType.DMA,
            pltpu.SemaphoreType.DMA,
            pltpu.SemaphoreType.DMA,
            pltpu.SemaphoreType.DMA,
            pltpu.SemaphoreType.DMA,
            pltpu.SemaphoreType.DMA,
            pltpu.SemaphoreType.DMA,
            pltpu.SemaphoreType.DMA,
            pltpu.SemaphoreType.DMA,
            pltpu.SemaphoreType.DMA,
            pltpu.SemaphoreType.DMA,
            pltpu.SemaphoreType.DMA,
        ],
    )(user_indices, item_indices, user_table.T, item_table.T)

    return pl.kernel(
        _dot_body,
        out_type=jax.ShapeDtypeStruct((BATCH,), jnp.float32),
        mesh=mesh,
        scratch_types=[
            pltpu.VMEM((B_PER_W * DIM,), jnp.float32),
            pltpu.VMEM((B_PER_W * DIM,), jnp.float32),
            pltpu.VMEM((B_PER_W,), jnp.float32),
        ],
    )(rowstage)


def kernel(user_indices, item_indices, user_table, item_table):
    return _wmf(user_indices.astype(jnp.int32), item_indices.astype(jnp.int32),
                user_table, item_table)
